# Initial kernel scaffold; baseline (speedup 1.0000x reference)
import jax
import jax.numpy as jnp
from jax.experimental import pallas as pl


def kernel(mask, image):
    raise NotImplementedError("write your pallas kernel here")


# speedup vs baseline: 63.3963x; 63.3963x over previous
import dataclasses
import functools

import jax
import jax.numpy as jnp
from jax import lax
from jax.experimental import pallas as pl
from jax.experimental.pallas import tpu as pltpu
from jax.experimental.pallas import tpu_sc as plsc

_THRESH = (0.07 - 0.5) / 0.5
_NBINS = 2048
_L = 16
_NC = 2
_NS = 16
_NW = _NC * _NS
_CHUNK = 8192

_mesh = plsc.VectorSubcoreMesh(core_axis_name="c", subcore_axis_name="s")

_cparams = pltpu.CompilerParams()
if "needs_layout_passes" in pltpu.CompilerParams.__dataclass_fields__:
    _cparams = dataclasses.replace(_cparams, needs_layout_passes=False)


def _hist_call(mask2d, img2d, n):
    grid = n // _CHUNK

    @functools.partial(
        pl.kernel,
        out_type=[
            jax.ShapeDtypeStruct((_NW, _NBINS), jnp.float32),
            jax.ShapeDtypeStruct((_NW, _NBINS), jnp.int32),
        ],
        mesh=_mesh,
        scratch_types=[
            pltpu.VMEM((_NBINS,), jnp.float32),
            pltpu.VMEM((_NBINS,), jnp.int32),
        ],
        compiler_params=_cparams,
    )
    def hist_kernel(mask_hbm, img_hbm, sums_hbm, counts_hbm, hs, hc):
        @pl.loop(0, _NBINS, step=_L)
        def _(i):
            hs[pl.ds(i, _L)] = jnp.zeros((_L,), jnp.float32)
            hc[pl.ds(i, _L)] = jnp.zeros((_L,), jnp.int32)

        ones = jnp.ones((_L,), jnp.int32)

        def body(m_vmem, v_vmem):
            @pl.loop(0, _CHUNK, step=_L)
            def _(i):
                lbl = m_vmem[0, pl.ds(i, _L)]
                val = v_vmem[0, pl.ds(i, _L)]
                plsc.addupdate_scatter(hs, [lbl], val)
                plsc.addupdate_scatter(hc, [lbl], ones)

        pltpu.emit_pipeline(
            body,
            grid=(grid,),
            in_specs=[
                pl.BlockSpec((1, _CHUNK), lambda i: (0, i)),
                pl.BlockSpec((1, _CHUNK), lambda i: (0, i)),
            ],
            out_specs=[],
            core_axis_name=("c", "s"),
            dimension_semantics=(pltpu.PARALLEL,),
        )(mask_hbm, img_hbm)

        wid = lax.axis_index("s") * _NC + lax.axis_index("c")
        pltpu.sync_copy(hs, sums_hbm.at[wid])
        pltpu.sync_copy(hc, counts_hbm.at[wid])

    return hist_kernel(mask2d, img2d)


def _remap_call(mask2d, sums, counts, n):
    grid = n // _CHUNK
    colblk = 512

    @functools.partial(
        pl.kernel,
        out_type=jax.ShapeDtypeStruct((1, n), jnp.int32),
        mesh=_mesh,
        scratch_types=[
            pltpu.VMEM((_NW, colblk), jnp.float32),
            pltpu.VMEM((_NW, colblk), jnp.int32),
            pltpu.VMEM((_NBINS,), jnp.int32),
        ],
        compiler_params=_cparams,
    )
    def remap_kernel(mask_hbm, sums_hbm, counts_hbm, out_hbm, sb, cb, remap):
        thresh = jnp.float32(_THRESH)

        @pl.loop(0, _NBINS // colblk)
        def _(b):
            pltpu.sync_copy(sums_hbm.at[:, pl.ds(b * colblk, colblk)], sb)
            pltpu.sync_copy(counts_hbm.at[:, pl.ds(b * colblk, colblk)], cb)

            @pl.loop(0, colblk, step=_L)
            def _(j):
                def row(r, carry):
                    s, c = carry
                    return (s + sb[r, pl.ds(j, _L)], c + cb[r, pl.ds(j, _L)])

                s, c = lax.fori_loop(
                    0, _NW, row,
                    (jnp.zeros((_L,), jnp.float32), jnp.zeros((_L,), jnp.int32)),
                )
                keep = s >= thresh * c.astype(jnp.float32)
                lbl = lax.iota(jnp.int32, _L) + (b * colblk + j)
                remap[pl.ds(b * colblk + j, _L)] = jnp.where(keep, lbl, 0)

        def body(m_vmem, o_vmem):
            @pl.loop(0, _CHUNK, step=_L)
            def _(i):
                lbl = m_vmem[0, pl.ds(i, _L)]
                o_vmem[0, pl.ds(i, _L)] = plsc.load_gather(remap, [lbl])

        pltpu.emit_pipeline(
            body,
            grid=(grid,),
            in_specs=[pl.BlockSpec((1, _CHUNK), lambda i: (0, i))],
            out_specs=[pl.BlockSpec((1, _CHUNK), lambda i: (0, i))],
            core_axis_name=("c", "s"),
            dimension_semantics=(pltpu.PARALLEL,),
        )(mask_hbm, out_hbm)

    return remap_kernel(mask2d, sums, counts)


@jax.jit
def kernel(mask, image):
    shape = mask.shape
    n = mask.size
    mask2d = mask.reshape(1, n)
    img2d = image.reshape(1, n)
    sums, counts = _hist_call(mask2d, img2d, n)
    new_mask = _remap_call(mask2d, sums, counts, n)
    return new_mask.reshape(shape)


# speedup vs baseline: 182.2802x; 2.8753x over previous
import dataclasses
import functools

import jax
import jax.numpy as jnp
from jax import lax
from jax.experimental import pallas as pl
from jax.experimental.pallas import tpu as pltpu
from jax.experimental.pallas import tpu_sc as plsc

_THRESH = (0.07 - 0.5) / 0.5
_NBINS = 2048
_L = 16
_NC = 2
_NS = 16
_NW = _NC * _NS
_H = 256
_SL = 64
_LN = 256

_mesh = plsc.VectorSubcoreMesh(core_axis_name="c", subcore_axis_name="s")

_cparams = pltpu.CompilerParams()
if "needs_layout_passes" in pltpu.CompilerParams.__dataclass_fields__:
    _cparams = dataclasses.replace(_cparams, needs_layout_passes=False)
_cparams = dataclasses.replace(_cparams, use_tc_tiling_on_sc=True)


def _hist_call(mask3d, img3d):
    @functools.partial(
        pl.kernel,
        out_type=[
            jax.ShapeDtypeStruct((_NW, _NBINS), jnp.float32),
            jax.ShapeDtypeStruct((_NW, _NBINS), jnp.int32),
        ],
        mesh=_mesh,
        scratch_types=[
            pltpu.VMEM((_NBINS,), jnp.float32),
            pltpu.VMEM((_NBINS,), jnp.int32),
        ],
        compiler_params=_cparams,
    )
    def hist_kernel(mask_hbm, img_hbm, sums_hbm, counts_hbm, hs, hc):
        @pl.loop(0, _NBINS, step=_L)
        def _(i):
            hs[pl.ds(i, _L)] = jnp.zeros((_L,), jnp.float32)
            hc[pl.ds(i, _L)] = jnp.zeros((_L,), jnp.int32)

        ones = jnp.ones((_L,), jnp.int32)

        def body(m_vmem, v_vmem):
            @pl.loop(0, _SL)
            def _(j):
                @pl.loop(0, _LN, step=_L, unroll=4)
                def _(k):
                    lbl = m_vmem[0, j, pl.ds(k, _L)]
                    val = v_vmem[0, j, pl.ds(k, _L)]
                    plsc.addupdate_scatter(hs, [lbl], val)
                    plsc.addupdate_scatter(hc, [lbl], ones)

        pltpu.emit_pipeline(
            body,
            grid=(_H,),
            in_specs=[
                pl.BlockSpec((1, _SL, _LN), lambda i: (i, 0, 0)),
                pl.BlockSpec((1, _SL, _LN), lambda i: (i, 0, 0)),
            ],
            out_specs=[],
            core_axis_name=("c", "s"),
            dimension_semantics=(pltpu.PARALLEL,),
        )(mask_hbm, img_hbm)

        wid = lax.axis_index("s") * _NC + lax.axis_index("c")
        pltpu.sync_copy(hs, sums_hbm.at[wid])
        pltpu.sync_copy(hc, counts_hbm.at[wid])

    return hist_kernel(mask3d, img3d)


def _remap_call(mask3d, sums, counts):
    seg = _NBINS // _NS

    @functools.partial(
        pl.kernel,
        out_type=jax.ShapeDtypeStruct((_H, _SL, _LN), jnp.int32),
        mesh=_mesh,
        scratch_types=[
            pltpu.VMEM((_NW, seg), jnp.float32),
            pltpu.VMEM((_NW, seg), jnp.int32),
            pltpu.VMEM((seg,), jnp.int32),
            pltpu.VMEM((_NBINS,), jnp.int32),
            pltpu.VMEM_SHARED((_NBINS,), jnp.int32),
        ],
        compiler_params=_cparams,
    )
    def remap_kernel(mask_hbm, sums_hbm, counts_hbm, out_hbm,
                     sb, cb, rseg, remap, shared_remap):
        thresh = jnp.float32(_THRESH)
        sid = lax.axis_index("s")

        pltpu.sync_copy(sums_hbm.at[:, pl.ds(sid * seg, seg)], sb)
        pltpu.sync_copy(counts_hbm.at[:, pl.ds(sid * seg, seg)], cb)

        @pl.loop(0, seg, step=_L)
        def _(j):
            def row(r, carry):
                s, c = carry
                return (s + sb[r, pl.ds(j, _L)], c + cb[r, pl.ds(j, _L)])

            s, c = lax.fori_loop(
                0, _NW, row,
                (jnp.zeros((_L,), jnp.float32), jnp.zeros((_L,), jnp.int32)),
            )
            keep = s >= thresh * c.astype(jnp.float32)
            lbl = lax.iota(jnp.int32, _L) + (sid * seg + j)
            rseg[pl.ds(j, _L)] = jnp.where(keep, lbl, 0)

        pltpu.sync_copy(rseg, shared_remap.at[pl.ds(sid * seg, seg)])
        plsc.subcore_barrier()
        pltpu.sync_copy(shared_remap, remap)

        def body(m_vmem, o_vmem):
            @pl.loop(0, _SL)
            def _(j):
                @pl.loop(0, _LN, step=_L, unroll=4)
                def _(k):
                    lbl = m_vmem[0, j, pl.ds(k, _L)]
                    o_vmem[0, j, pl.ds(k, _L)] = plsc.load_gather(remap, [lbl])

        pltpu.emit_pipeline(
            body,
            grid=(_H,),
            in_specs=[pl.BlockSpec((1, _SL, _LN), lambda i: (i, 0, 0))],
            out_specs=[pl.BlockSpec((1, _SL, _LN), lambda i: (i, 0, 0))],
            core_axis_name=("c", "s"),
            dimension_semantics=(pltpu.PARALLEL,),
        )(mask_hbm, out_hbm)

    return remap_kernel(mask3d, sums, counts)


@jax.jit
def kernel(mask, image):
    shape = mask.shape
    mask3d = mask.transpose((0, 1, 2, 4, 3)).reshape(_H, _SL, _LN)
    img3d = image.transpose((0, 1, 2, 4, 3)).reshape(_H, _SL, _LN)
    sums, counts = _hist_call(mask3d, img3d)
    new3d = _remap_call(mask3d, sums, counts)
    return new3d.reshape(1, 1, _H, _SL, _LN).transpose((0, 1, 2, 4, 3)).reshape(shape)


# speedup vs baseline: 182.3176x; 1.0002x over previous
import dataclasses
import functools

import jax
import jax.numpy as jnp
from jax import lax
from jax.experimental import pallas as pl
from jax.experimental.pallas import tpu as pltpu
from jax.experimental.pallas import tpu_sc as plsc

_THRESH = (0.07 - 0.5) / 0.5
_NBINS = 2048
_L = 16
_NC = 2
_NS = 16
_NW = _NC * _NS
_H = 256
_SL = 64
_LN = 256

_mesh = plsc.VectorSubcoreMesh(core_axis_name="c", subcore_axis_name="s")

_cparams = pltpu.CompilerParams()
if "needs_layout_passes" in pltpu.CompilerParams.__dataclass_fields__:
    _cparams = dataclasses.replace(_cparams, needs_layout_passes=False)
_cparams = dataclasses.replace(_cparams, use_tc_tiling_on_sc=True)


def _hist_call(mask3d, img3d):
    @functools.partial(
        pl.kernel,
        out_type=[
            jax.ShapeDtypeStruct((_NW, _NBINS), jnp.float32),
            jax.ShapeDtypeStruct((_NW, _NBINS), jnp.int32),
        ],
        mesh=_mesh,
        scratch_types=[
            pltpu.VMEM((_NBINS,), jnp.float32),
            pltpu.VMEM((_NBINS,), jnp.int32),
        ],
        compiler_params=_cparams,
    )
    def hist_kernel(mask_hbm, img_hbm, sums_hbm, counts_hbm, hs, hc):
        @pl.loop(0, _NBINS, step=_L)
        def _(i):
            hs[pl.ds(i, _L)] = jnp.zeros((_L,), jnp.float32)
            hc[pl.ds(i, _L)] = jnp.zeros((_L,), jnp.int32)

        ones = jnp.ones((_L,), jnp.int32)

        def body(m_vmem, v_vmem):
            @pl.loop(0, _SL)
            def _(j):
                @pl.loop(0, _LN, step=_L, unroll=16)
                def _(k):
                    lbl = m_vmem[0, j, pl.ds(k, _L)]
                    val = v_vmem[0, j, pl.ds(k, _L)]
                    plsc.addupdate_scatter(hs, [lbl], val)
                    plsc.addupdate_scatter(hc, [lbl], ones)

        pltpu.emit_pipeline(
            body,
            grid=(_H,),
            in_specs=[
                pl.BlockSpec((1, _SL, _LN), lambda i: (i, 0, 0)),
                pl.BlockSpec((1, _SL, _LN), lambda i: (i, 0, 0)),
            ],
            out_specs=[],
            core_axis_name=("c", "s"),
            dimension_semantics=(pltpu.PARALLEL,),
        )(mask_hbm, img_hbm)

        wid = lax.axis_index("s") * _NC + lax.axis_index("c")
        pltpu.sync_copy(hs, sums_hbm.at[wid])
        pltpu.sync_copy(hc, counts_hbm.at[wid])

    return hist_kernel(mask3d, img3d)


def _remap_call(mask3d, sums, counts):
    seg = _NBINS // _NS

    @functools.partial(
        pl.kernel,
        out_type=jax.ShapeDtypeStruct((_H, _SL, _LN), jnp.int32),
        mesh=_mesh,
        scratch_types=[
            pltpu.VMEM((_NW, seg), jnp.float32),
            pltpu.VMEM((_NW, seg), jnp.int32),
            pltpu.VMEM((seg,), jnp.int32),
            pltpu.VMEM((_NBINS,), jnp.int32),
            pltpu.VMEM_SHARED((_NBINS,), jnp.int32),
        ],
        compiler_params=_cparams,
    )
    def remap_kernel(mask_hbm, sums_hbm, counts_hbm, out_hbm,
                     sb, cb, rseg, remap, shared_remap):
        thresh = jnp.float32(_THRESH)
        sid = lax.axis_index("s")

        pltpu.sync_copy(sums_hbm.at[:, pl.ds(sid * seg, seg)], sb)
        pltpu.sync_copy(counts_hbm.at[:, pl.ds(sid * seg, seg)], cb)

        @pl.loop(0, seg, step=_L)
        def _(j):
            def row(r, carry):
                s, c = carry
                return (s + sb[r, pl.ds(j, _L)], c + cb[r, pl.ds(j, _L)])

            s, c = lax.fori_loop(
                0, _NW, row,
                (jnp.zeros((_L,), jnp.float32), jnp.zeros((_L,), jnp.int32)),
            )
            keep = s >= thresh * c.astype(jnp.float32)
            lbl = lax.iota(jnp.int32, _L) + (sid * seg + j)
            rseg[pl.ds(j, _L)] = jnp.where(keep, lbl, 0)

        pltpu.sync_copy(rseg, shared_remap.at[pl.ds(sid * seg, seg)])
        plsc.subcore_barrier()
        pltpu.sync_copy(shared_remap, remap)

        def body(m_vmem, o_vmem):
            @pl.loop(0, _SL)
            def _(j):
                @pl.loop(0, _LN, step=_L, unroll=16)
                def _(k):
                    lbl = m_vmem[0, j, pl.ds(k, _L)]
                    o_vmem[0, j, pl.ds(k, _L)] = plsc.load_gather(remap, [lbl])

        pltpu.emit_pipeline(
            body,
            grid=(_H,),
            in_specs=[pl.BlockSpec((1, _SL, _LN), lambda i: (i, 0, 0))],
            out_specs=[pl.BlockSpec((1, _SL, _LN), lambda i: (i, 0, 0))],
            core_axis_name=("c", "s"),
            dimension_semantics=(pltpu.PARALLEL,),
        )(mask_hbm, out_hbm)

    return remap_kernel(mask3d, sums, counts)


@jax.jit
def kernel(mask, image):
    shape = mask.shape
    mask3d = mask.transpose((0, 1, 2, 4, 3)).reshape(_H, _SL, _LN)
    img3d = image.transpose((0, 1, 2, 4, 3)).reshape(_H, _SL, _LN)
    sums, counts = _hist_call(mask3d, img3d)
    new3d = _remap_call(mask3d, sums, counts)
    return new3d.reshape(1, 1, _H, _SL, _LN).transpose((0, 1, 2, 4, 3)).reshape(shape)


# speedup vs baseline: 321.9449x; 1.7658x over previous
import dataclasses
import functools

import jax
import jax.numpy as jnp
from jax import lax
from jax.experimental import pallas as pl
from jax.experimental.pallas import tpu as pltpu
from jax.experimental.pallas import tpu_sc as plsc

_THRESH = (0.07 - 0.5) / 0.5
_NBINS = 2048
_L = 16
_NC = 2
_NS = 16
_NW = _NC * _NS
_H = 256
_SL = 64
_LN = 256

_mesh = plsc.VectorSubcoreMesh(core_axis_name="c", subcore_axis_name="s")

_cparams = pltpu.CompilerParams()
if "needs_layout_passes" in pltpu.CompilerParams.__dataclass_fields__:
    _cparams = dataclasses.replace(_cparams, needs_layout_passes=False)
_cparams = dataclasses.replace(_cparams, use_tc_tiling_on_sc=True)


def _hist_call(mask3d, img3d):
    @functools.partial(
        pl.kernel,
        out_type=jax.ShapeDtypeStruct((_NW, _NBINS), jnp.float32),
        mesh=_mesh,
        scratch_types=[pltpu.VMEM((_NBINS,), jnp.float32)],
        compiler_params=_cparams,
    )
    def hist_kernel(mask_hbm, img_hbm, acc_hbm, ha):
        @pl.loop(0, _NBINS, step=_L)
        def _(i):
            ha[pl.ds(i, _L)] = jnp.zeros((_L,), jnp.float32)

        shift = jnp.float32(-_THRESH)
        niter = _SL * _LN // _L

        def _ld(ref, i):
            return ref[0, i >> 4, pl.ds((i & 15) * _L, _L)]

        def body(m_vmem, v_vmem):
            def step(i, carry):
                lbl, val = carry
                nlbl = _ld(m_vmem, i + 1)
                nval = _ld(v_vmem, i + 1)
                plsc.addupdate_scatter(ha, [lbl], val + shift)
                return (nlbl, nval)

            lbl, val = pl.loop(
                0, niter - 1, unroll=8,
                init_carry=(_ld(m_vmem, 0), _ld(v_vmem, 0)),
            )(step)
            plsc.addupdate_scatter(ha, [lbl], val + shift)

        pltpu.emit_pipeline(
            body,
            grid=(_H,),
            in_specs=[
                pl.BlockSpec((1, _SL, _LN), lambda i: (i, 0, 0)),
                pl.BlockSpec((1, _SL, _LN), lambda i: (i, 0, 0)),
            ],
            out_specs=[],
            core_axis_name=("c", "s"),
            dimension_semantics=(pltpu.PARALLEL,),
        )(mask_hbm, img_hbm)

        wid = lax.axis_index("s") * _NC + lax.axis_index("c")
        pltpu.sync_copy(ha, acc_hbm.at[wid])

    return hist_kernel(mask3d, img3d)


def _remap_call(mask3d, acc):
    seg = _NBINS // _NS

    @functools.partial(
        pl.kernel,
        out_type=jax.ShapeDtypeStruct((_H, _SL, _LN), jnp.int32),
        mesh=_mesh,
        scratch_types=[
            pltpu.VMEM((_NW, seg), jnp.float32),
            pltpu.VMEM((seg,), jnp.int32),
            pltpu.VMEM((_NBINS,), jnp.int32),
            pltpu.VMEM_SHARED((_NBINS,), jnp.int32),
        ],
        compiler_params=_cparams,
    )
    def remap_kernel(mask_hbm, acc_hbm, out_hbm, ab, rseg, remap, shared_remap):
        sid = lax.axis_index("s")

        pltpu.sync_copy(acc_hbm.at[:, pl.ds(sid * seg, seg)], ab)

        @pl.loop(0, seg, step=_L)
        def _(j):
            def row(r, s):
                return s + ab[r, pl.ds(j, _L)]

            s = lax.fori_loop(0, _NW, row, jnp.zeros((_L,), jnp.float32))
            keep = s >= jnp.float32(0.0)
            lbl = lax.iota(jnp.int32, _L) + (sid * seg + j)
            rseg[pl.ds(j, _L)] = jnp.where(keep, lbl, 0)

        pltpu.sync_copy(rseg, shared_remap.at[pl.ds(sid * seg, seg)])
        plsc.subcore_barrier()
        pltpu.sync_copy(shared_remap, remap)

        niter = _SL * _LN // _L

        def _ld(ref, i):
            return ref[0, i >> 4, pl.ds((i & 15) * _L, _L)]

        def _st(ref, i, x):
            ref[0, i >> 4, pl.ds((i & 15) * _L, _L)] = x

        def body(m_vmem, o_vmem):
            l0 = _ld(m_vmem, 0)
            l1 = _ld(m_vmem, 1)
            l2 = _ld(m_vmem, 2)
            g0 = plsc.load_gather(remap, [l0])

            def step(i, carry):
                g, la, lb = carry
                _st(o_vmem, i, g)
                gn = plsc.load_gather(remap, [la])
                ln = _ld(m_vmem, i + 3)
                return (gn, lb, ln)

            g, la, lb = pl.loop(
                0, niter - 3, unroll=8, init_carry=(g0, l1, l2)
            )(step)
            _st(o_vmem, niter - 3, g)
            g = plsc.load_gather(remap, [la])
            _st(o_vmem, niter - 2, g)
            g = plsc.load_gather(remap, [lb])
            _st(o_vmem, niter - 1, g)

        pltpu.emit_pipeline(
            body,
            grid=(_H,),
            in_specs=[pl.BlockSpec((1, _SL, _LN), lambda i: (i, 0, 0))],
            out_specs=[pl.BlockSpec((1, _SL, _LN), lambda i: (i, 0, 0))],
            core_axis_name=("c", "s"),
            dimension_semantics=(pltpu.PARALLEL,),
        )(mask_hbm, out_hbm)

    return remap_kernel(mask3d, acc)


@jax.jit
def kernel(mask, image):
    shape = mask.shape
    mask3d = mask.transpose((0, 1, 2, 4, 3)).reshape(_H, _SL, _LN)
    img3d = image.transpose((0, 1, 2, 4, 3)).reshape(_H, _SL, _LN)
    acc = _hist_call(mask3d, img3d)
    new3d = _remap_call(mask3d, acc)
    return new3d.reshape(1, 1, _H, _SL, _LN).transpose((0, 1, 2, 4, 3)).reshape(shape)


# speedup vs baseline: 376.6087x; 1.1698x over previous
import dataclasses
import functools

import jax
import jax.numpy as jnp
from jax import lax
from jax.experimental import pallas as pl
from jax.experimental.pallas import tpu as pltpu
from jax.experimental.pallas import tpu_sc as plsc

_THRESH = (0.07 - 0.5) / 0.5
_NBINS = 2048
_L = 16
_NC = 2
_NS = 16
_NW = _NC * _NS
_H = 256
_SL = 64
_LN = 256

_mesh = plsc.VectorSubcoreMesh(core_axis_name="c", subcore_axis_name="s")

_cparams = pltpu.CompilerParams()
if "needs_layout_passes" in pltpu.CompilerParams.__dataclass_fields__:
    _cparams = dataclasses.replace(_cparams, needs_layout_passes=False)
_cparams = dataclasses.replace(_cparams, use_tc_tiling_on_sc=True)


def _hist_call(mask3d, img3d):
    @functools.partial(
        pl.kernel,
        out_type=jax.ShapeDtypeStruct((_NW, _NBINS), jnp.float32),
        mesh=_mesh,
        scratch_types=[pltpu.VMEM((_NBINS,), jnp.float32)],
        compiler_params=_cparams,
    )
    def hist_kernel(mask_hbm, img_hbm, acc_hbm, ha):
        @pl.loop(0, _NBINS, step=_L)
        def _(i):
            ha[pl.ds(i, _L)] = jnp.zeros((_L,), jnp.float32)

        shift = jnp.float32(-_THRESH)
        niter = _SL * _LN // _L

        def _ld(ref, i):
            return ref[0, i >> 4, pl.ds((i & 15) * _L, _L)]

        def body(m_vmem, v_vmem):
            la = _ld(m_vmem, 0)
            va = _ld(v_vmem, 0) + shift
            lb = _ld(m_vmem, 1)
            vb = _ld(v_vmem, 1) + shift

            def step(i, carry):
                la, va, lb, vb = carry
                plsc.addupdate_scatter(ha, [la], va)
                ln = _ld(m_vmem, i + 2)
                vn = _ld(v_vmem, i + 2) + shift
                return (lb, vb, ln, vn)

            la, va, lb, vb = pl.loop(
                0, niter - 2, unroll=8, init_carry=(la, va, lb, vb)
            )(step)
            plsc.addupdate_scatter(ha, [la], va)
            plsc.addupdate_scatter(ha, [lb], vb)

        pltpu.emit_pipeline(
            body,
            grid=(_H,),
            in_specs=[
                pl.BlockSpec((1, _SL, _LN), lambda i: (i, 0, 0)),
                pl.BlockSpec((1, _SL, _LN), lambda i: (i, 0, 0)),
            ],
            out_specs=[],
            core_axis_name=("c", "s"),
            dimension_semantics=(pltpu.PARALLEL,),
        )(mask_hbm, img_hbm)

        wid = lax.axis_index("s") * _NC + lax.axis_index("c")
        pltpu.sync_copy(ha, acc_hbm.at[wid])

    return hist_kernel(mask3d, img3d)


def _remap_call(mask3d, acc):
    seg = _NBINS // _NS

    @functools.partial(
        pl.kernel,
        out_type=jax.ShapeDtypeStruct((_H, _SL, _LN), jnp.int32),
        mesh=_mesh,
        scratch_types=[
            pltpu.VMEM((_NW, seg), jnp.float32),
            pltpu.VMEM((seg,), jnp.int32),
            pltpu.VMEM((_NBINS,), jnp.int32),
            pltpu.VMEM_SHARED((_NBINS,), jnp.int32),
        ],
        compiler_params=_cparams,
    )
    def remap_kernel(mask_hbm, acc_hbm, out_hbm, ab, rseg, remap, shared_remap):
        sid = lax.axis_index("s")

        pltpu.sync_copy(acc_hbm.at[:, pl.ds(sid * seg, seg)], ab)

        @pl.loop(0, seg, step=_L)
        def _(j):
            def row(r, s):
                return s + ab[r, pl.ds(j, _L)]

            s = lax.fori_loop(0, _NW, row, jnp.zeros((_L,), jnp.float32))
            keep = s >= jnp.float32(0.0)
            lbl = lax.iota(jnp.int32, _L) + (sid * seg + j)
            rseg[pl.ds(j, _L)] = jnp.where(keep, lbl, 0)

        pltpu.sync_copy(rseg, shared_remap.at[pl.ds(sid * seg, seg)])
        plsc.subcore_barrier()
        pltpu.sync_copy(shared_remap, remap)

        niter = _SL * _LN // _L

        def _ld(ref, i):
            return ref[0, i >> 4, pl.ds((i & 15) * _L, _L)]

        def _st(ref, i, x):
            ref[0, i >> 4, pl.ds((i & 15) * _L, _L)] = x

        def body(m_vmem, o_vmem):
            l0 = _ld(m_vmem, 0)
            l1 = _ld(m_vmem, 1)
            l2 = _ld(m_vmem, 2)
            l3 = _ld(m_vmem, 3)
            ga = plsc.load_gather(remap, [l0])
            gb = plsc.load_gather(remap, [l1])

            def step(i, carry):
                ga, gb, la, lb = carry
                _st(o_vmem, i, ga)
                gn = plsc.load_gather(remap, [la])
                ln = _ld(m_vmem, i + 4)
                return (gb, gn, lb, ln)

            ga, gb, la, lb = pl.loop(
                0, niter - 4, unroll=8, init_carry=(ga, gb, l2, l3)
            )(step)
            _st(o_vmem, niter - 4, ga)
            _st(o_vmem, niter - 3, gb)
            g = plsc.load_gather(remap, [la])
            _st(o_vmem, niter - 2, g)
            g = plsc.load_gather(remap, [lb])
            _st(o_vmem, niter - 1, g)

        pltpu.emit_pipeline(
            body,
            grid=(_H,),
            in_specs=[pl.BlockSpec((1, _SL, _LN), lambda i: (i, 0, 0))],
            out_specs=[pl.BlockSpec((1, _SL, _LN), lambda i: (i, 0, 0))],
            core_axis_name=("c", "s"),
            dimension_semantics=(pltpu.PARALLEL,),
        )(mask_hbm, out_hbm)

    return remap_kernel(mask3d, acc)


@jax.jit
def kernel(mask, image):
    shape = mask.shape
    mask3d = mask.transpose((0, 1, 2, 4, 3)).reshape(_H, _SL, _LN)
    img3d = image.transpose((0, 1, 2, 4, 3)).reshape(_H, _SL, _LN)
    acc = _hist_call(mask3d, img3d)
    new3d = _remap_call(mask3d, acc)
    return new3d.reshape(1, 1, _H, _SL, _LN).transpose((0, 1, 2, 4, 3)).reshape(shape)


# speedup vs baseline: 380.4649x; 1.0102x over previous
import dataclasses
import functools

import jax
import jax.numpy as jnp
from jax import lax
from jax.experimental import pallas as pl
from jax.experimental.pallas import tpu as pltpu
from jax.experimental.pallas import tpu_sc as plsc

_THRESH = (0.07 - 0.5) / 0.5
_NBINS = 2048
_L = 16
_NC = 2
_NS = 16
_NW = _NC * _NS
_H = 256
_SL = 64
_LN = 256

_mesh = plsc.VectorSubcoreMesh(core_axis_name="c", subcore_axis_name="s")

_cparams = pltpu.CompilerParams()
if "needs_layout_passes" in pltpu.CompilerParams.__dataclass_fields__:
    _cparams = dataclasses.replace(_cparams, needs_layout_passes=False)
_cparams = dataclasses.replace(_cparams, use_tc_tiling_on_sc=True)


def _hist_call(mask3d, img3d):
    @functools.partial(
        pl.kernel,
        out_type=jax.ShapeDtypeStruct((_NW, _NBINS), jnp.float32),
        mesh=_mesh,
        scratch_types=[pltpu.VMEM((_NBINS,), jnp.float32)],
        compiler_params=_cparams,
    )
    def hist_kernel(mask_hbm, img_hbm, acc_hbm, ha):
        @pl.loop(0, _NBINS, step=_L)
        def _(i):
            ha[pl.ds(i, _L)] = jnp.zeros((_L,), jnp.float32)

        shift = jnp.float32(-_THRESH)
        niter = _SL * _LN // _L

        def _ld(ref, i):
            return ref[0, i >> 4, pl.ds((i & 15) * _L, _L)]

        def body(m_vmem, v_vmem):
            def step(i, carry):
                lbl, val = carry
                nlbl = _ld(m_vmem, i + 1)
                nval = _ld(v_vmem, i + 1)
                plsc.addupdate_scatter(ha, [lbl], val + shift)
                return (nlbl, nval)

            lbl, val = pl.loop(
                0, niter - 1, unroll=8,
                init_carry=(_ld(m_vmem, 0), _ld(v_vmem, 0)),
            )(step)
            plsc.addupdate_scatter(ha, [lbl], val + shift)

        pltpu.emit_pipeline(
            body,
            grid=(_H,),
            in_specs=[
                pl.BlockSpec((1, _SL, _LN), lambda i: (i, 0, 0)),
                pl.BlockSpec((1, _SL, _LN), lambda i: (i, 0, 0)),
            ],
            out_specs=[],
            core_axis_name=("c", "s"),
            dimension_semantics=(pltpu.PARALLEL,),
        )(mask_hbm, img_hbm)

        wid = lax.axis_index("s") * _NC + lax.axis_index("c")
        pltpu.sync_copy(ha, acc_hbm.at[wid])

    return hist_kernel(mask3d, img3d)


def _remap_call(mask3d, acc):
    seg = _NBINS // _NS

    @functools.partial(
        pl.kernel,
        out_type=jax.ShapeDtypeStruct((_H, _SL, _LN), jnp.int32),
        mesh=_mesh,
        scratch_types=[
            pltpu.VMEM((_NW, seg), jnp.float32),
            pltpu.VMEM((seg,), jnp.int32),
            pltpu.VMEM((_NBINS,), jnp.int32),
            pltpu.VMEM_SHARED((_NBINS,), jnp.int32),
        ],
        compiler_params=_cparams,
    )
    def remap_kernel(mask_hbm, acc_hbm, out_hbm, ab, rseg, remap, shared_remap):
        sid = lax.axis_index("s")

        pltpu.sync_copy(acc_hbm.at[:, pl.ds(sid * seg, seg)], ab)

        @pl.loop(0, seg, step=_L)
        def _(j):
            def row(r, s):
                return s + ab[r, pl.ds(j, _L)]

            s = lax.fori_loop(0, _NW, row, jnp.zeros((_L,), jnp.float32))
            keep = s >= jnp.float32(0.0)
            lbl = lax.iota(jnp.int32, _L) + (sid * seg + j)
            rseg[pl.ds(j, _L)] = jnp.where(keep, lbl, 0)

        pltpu.sync_copy(rseg, shared_remap.at[pl.ds(sid * seg, seg)])
        plsc.subcore_barrier()
        pltpu.sync_copy(shared_remap, remap)

        niter = _SL * _LN // _L

        def _ld(ref, i):
            return ref[0, i >> 4, pl.ds((i & 15) * _L, _L)]

        def _st(ref, i, x):
            ref[0, i >> 4, pl.ds((i & 15) * _L, _L)] = x

        def body(m_vmem, o_vmem):
            l0 = _ld(m_vmem, 0)
            l1 = _ld(m_vmem, 1)
            l2 = _ld(m_vmem, 2)
            l3 = _ld(m_vmem, 3)
            ga = plsc.load_gather(remap, [l0])
            gb = plsc.load_gather(remap, [l1])

            def step(i, carry):
                ga, gb, la, lb = carry
                _st(o_vmem, i, ga)
                gn = plsc.load_gather(remap, [la])
                ln = _ld(m_vmem, i + 4)
                return (gb, gn, lb, ln)

            ga, gb, la, lb = pl.loop(
                0, niter - 4, unroll=8, init_carry=(ga, gb, l2, l3)
            )(step)
            _st(o_vmem, niter - 4, ga)
            _st(o_vmem, niter - 3, gb)
            g = plsc.load_gather(remap, [la])
            _st(o_vmem, niter - 2, g)
            g = plsc.load_gather(remap, [lb])
            _st(o_vmem, niter - 1, g)

        pltpu.emit_pipeline(
            body,
            grid=(_H,),
            in_specs=[pl.BlockSpec((1, _SL, _LN), lambda i: (i, 0, 0))],
            out_specs=[pl.BlockSpec((1, _SL, _LN), lambda i: (i, 0, 0))],
            core_axis_name=("c", "s"),
            dimension_semantics=(pltpu.PARALLEL,),
        )(mask_hbm, out_hbm)

    return remap_kernel(mask3d, acc)


@jax.jit
def kernel(mask, image):
    shape = mask.shape
    mask3d = mask.transpose((0, 1, 2, 4, 3)).reshape(_H, _SL, _LN)
    img3d = image.transpose((0, 1, 2, 4, 3)).reshape(_H, _SL, _LN)
    acc = _hist_call(mask3d, img3d)
    new3d = _remap_call(mask3d, acc)
    return new3d.reshape(1, 1, _H, _SL, _LN).transpose((0, 1, 2, 4, 3)).reshape(shape)
